# baseline (device time: 68783 ns/iter reference)
import jax
import jax.numpy as jnp
from jax import lax
from jax.experimental import pallas as pl
from jax.experimental.pallas import tpu as pltpu

N_DEV = 8
M = 2048
N = 2048
CH = N // N_DEV

NS = 4
HALVES = (
    ((0, 176), (176, 176), (352, 176), (528, 160)),
    ((688, 176), (864, 160), (1024, 176), (1200, 160)),
    ((1360, 176), (1536, 176), (1712, 176), (1888, 160)),
)
AXES_RS = ((0, 1, 2), (1, 2, 0), (2, 0, 1))
CSLOT = (0, 1024, 1536)


def kernel(x, w_mat):
    def body(x_ref, w_ref, out_ref, abuf, rbuf, gbuf, send_sems, recv_sems):
        p = lax.axis_index("i")
        q = p ^ ((p >> 1) & 1)
        qb = [q & 1, (q >> 1) & 1, (q >> 2) & 1]

        def to_p(qq):
            return qq ^ ((qq >> 1) & 1)

        parts = [to_p(q ^ 1), to_p(q ^ 2), to_p(q ^ 4)]

        barrier = pltpu.get_barrier_semaphore()
        for nbr in parts:
            pl.semaphore_signal(
                barrier, inc=1, device_id=(nbr,),
                device_id_type=pl.DeviceIdType.MESH,
            )
        pl.semaphore_wait(barrier, 3)

        ck = [0, 0, 0]
        rs_cols = []
        for j in range(3):
            row = []
            for t in range(3):
                bit = AXES_RS[t][j]
                ncols = 1024 >> j
                csend = ck[t] + (1 - qb[bit]) * ncols
                ck[t] = ck[t] + qb[bit] * ncols
                row.append((bit, csend, ck[t], ncols))
            rs_cols.append(row)

        def rcopy(src_ref, dst_ref, r0, nr, csrc, cdst, ncols, bit, sem):
            return pltpu.make_async_remote_copy(
                src_ref=src_ref.at[pl.ds(r0, nr), pl.ds(csrc, ncols)],
                dst_ref=dst_ref.at[pl.ds(r0, nr), pl.ds(cdst, ncols)],
                send_sem=send_sems.at[sem],
                recv_sem=recv_sems.at[sem],
                device_id=(parts[bit],),
                device_id_type=pl.DeviceIdType.MESH,
            )

        def rs_send(j, t, h):
            bit, csend, _, ncols = rs_cols[j][t]
            r0, nr = HALVES[t][h]
            sem = (j * 3 + t) * NS + h
            r = rcopy(abuf, rbuf, r0, nr, csend, CSLOT[j], ncols, bit, sem)
            r.start()
            return r

        def gemm_tile(r0, nr, c0, nc):
            abuf[pl.ds(r0, nr), pl.ds(c0, nc)] = jnp.dot(
                x_ref[pl.ds(r0, nr), :],
                w_ref[:, pl.ds(c0, nc)],
                preferred_element_type=jnp.float32,
            ).astype(jnp.bfloat16)

        rd = {}
        for h in range(NS):
            for t in range(3):
                _, csend, ckeep0, _ = rs_cols[0][t]
                r0, nr = HALVES[t][h]
                gemm_tile(r0, nr, csend, 1024)
                rd[(0, t, h)] = rs_send(0, t, h)
        for h in range(NS):
            for t in range(3):
                _, _, ckeep0, _ = rs_cols[0][t]
                r0, nr = HALVES[t][h]
                gemm_tile(r0, nr, ckeep0, 1024)

        for j in range(3):
            for h in range(NS):
                for t in range(3):
                    bit, csend, ckj, ncols = rs_cols[j][t]
                    r0, nr = HALVES[t][h]
                    rd[(j, t, h)].wait()
                    abuf[pl.ds(r0, nr), pl.ds(ckj, ncols)] = (
                        abuf[pl.ds(r0, nr), pl.ds(ckj, ncols)]
                        + rbuf[pl.ds(r0, nr), pl.ds(CSLOT[j], ncols)]
                    )
                    if j < 2:
                        rd[(j + 1, t, h)] = rs_send(j + 1, t, h)
                    else:
                        y = abuf[
                            pl.ds(r0, nr), pl.ds(ck[t], CH)
                        ].astype(jnp.float32)
                        sil = y * jax.nn.sigmoid(y)
                        out_ref[pl.ds(r0, nr), pl.ds(ck[t], CH)] = sil
                        gbuf[pl.ds(r0, nr), pl.ds(ck[t], CH)] = sil.astype(
                            jnp.bfloat16
                        )
                        bit0 = AXES_RS[t][2]
                        sem = (3 * 3 + t) * NS + h
                        r = rcopy(
                            gbuf, gbuf, r0, nr, ck[t], ck[t], CH, bit0, sem
                        )
                        r.start()
                        rd[(3, t, h)] = r

        own = [(ck[t], CH) for t in range(3)]
        for j in range(3):
            new_own = list(own)
            for h in range(NS):
                for t in range(3):
                    bit = AXES_RS[t][2 - j]
                    co, nc = own[t]
                    parent = co - qb[bit] * nc
                    rco = parent + (1 - qb[bit]) * nc
                    r0, nr = HALVES[t][h]
                    rd[(3 + j, t, h)].wait()
                    out_ref[pl.ds(r0, nr), pl.ds(rco, nc)] = gbuf[
                        pl.ds(r0, nr), pl.ds(rco, nc)
                    ].astype(jnp.float32)
                    new_own[t] = (parent, nc * 2)
                    if j < 2:
                        bitn = AXES_RS[t][2 - (j + 1)]
                        sem = ((3 + j + 1) * 3 + t) * NS + h
                        r = rcopy(
                            gbuf, gbuf, r0, nr, parent, parent,
                            nc * 2, bitn, sem,
                        )
                        r.start()
                        rd[(3 + j + 1, t, h)] = r
            own = new_own

    return pl.pallas_call(
        body,
        out_shape=jax.ShapeDtypeStruct((M, N), jnp.float32),
        in_specs=[
            pl.BlockSpec(memory_space=pltpu.VMEM),
            pl.BlockSpec(memory_space=pltpu.VMEM),
        ],
        out_specs=pl.BlockSpec(memory_space=pltpu.VMEM),
        scratch_shapes=[
            pltpu.VMEM((M, N), jnp.bfloat16),
            pltpu.VMEM((M, 1792), jnp.bfloat16),
            pltpu.VMEM((M, N), jnp.bfloat16),
            pltpu.SemaphoreType.DMA((18 * NS,)),
            pltpu.SemaphoreType.DMA((18 * NS,)),
        ],
        compiler_params=pltpu.CompilerParams(collective_id=0),
    )(x, w_mat)
